# baseline (speedup 1.0000x reference)
import jax
import jax.numpy as jnp
from jax import lax
from jax.experimental import pallas as pl
from jax.experimental.pallas import tpu as pltpu
from jax.experimental.pallas import tpu_sc as plsc

B, N, M = 4, 4096, 4096
SCX = 256
R = N - SCX
NW = 32
WPB = NW // B
CH = M // WPB
SCPW = SCX // WPB
L = 16
C0 = R // L
NC = M // L


def _limbs(v):
    h = v.astype(jnp.bfloat16)
    r1 = v - h.astype(jnp.float32)
    m = r1.astype(jnp.bfloat16)
    l = (r1 - m.astype(jnp.float32)).astype(jnp.bfloat16)
    return h, m, l


def _tc_body(x_ref, y_ref, xsum_ref, ymin_ref):
    b = pl.program_id(0)

    x = x_ref[0]
    y = y_ref[0]
    xb = x.astype(jnp.bfloat16)
    w = (y.astype(jnp.bfloat16)) * jnp.bfloat16(-2.0)

    xsq = jnp.sum(x * x, axis=1, keepdims=True)
    ysq = jnp.sum(y * y, axis=0, keepdims=True)
    xh, xm, xl = _limbs(xsq)
    yh, ym, yl = _limbs(ysq)

    ones_x = jnp.ones((R, 3), jnp.bfloat16)
    ones_y = jnp.ones((3, M), jnp.bfloat16)
    xmat = jnp.concatenate([xb, ones_x, xh, xm, xl], axis=1)
    wmat = jnp.concatenate([w, yh, ym, yl, ones_y], axis=0)

    g = jax.lax.dot_general(xmat, wmat, (((1,), (0,)), ((), ())),
                            preferred_element_type=jnp.float32)

    row_min = jnp.maximum(jnp.min(g, axis=1), 0.0)
    ymin_ref[0] = jnp.min(g, axis=0, keepdims=True)

    @pl.when(b == 0)
    def _zero_out():
        xsum_ref[...] = jnp.zeros((1, 1), jnp.float32)

    xsum_ref[...] = xsum_ref[...] + jnp.sum(row_min)


def _tc_call(xyz1, yT):
    return pl.pallas_call(
        _tc_body,
        grid=(B,),
        in_specs=[
            pl.BlockSpec((1, R, 3), lambda b: (b, 0, 0)),
            pl.BlockSpec((1, 3, M), lambda b: (b, 0, 0)),
        ],
        out_specs=[
            pl.BlockSpec((1, 1), lambda b: (0, 0)),
            pl.BlockSpec((1, 1, M), lambda b: (b, 0, 0)),
        ],
        out_shape=[
            jax.ShapeDtypeStruct((1, 1), jnp.float32),
            jax.ShapeDtypeStruct((B, 1, M), jnp.float32),
        ],
    )(xyz1, yT)


def _round_bf16(v):
    u = lax.bitcast_convert_type(v, jnp.int32)
    lsb = lax.shift_right_logical(u, 16) & 1
    u2 = (u + 0x7FFF + lsb) & jnp.int32(-65536)
    return lax.bitcast_convert_type(u2, jnp.float32)


def _precompute(orig_ref, sq_ref, rnd_ref, m2_ref, c_lo, c_hi):
    def body(c, _):
        sl = pl.ds(c * L, L)
        v0 = orig_ref[0, sl]
        v1 = orig_ref[1, sl]
        v2 = orig_ref[2, sl]
        sq_ref[sl] = v0 * v0 + v1 * v1 + v2 * v2
        r0 = _round_bf16(v0)
        r1 = _round_bf16(v1)
        r2 = _round_bf16(v2)
        rnd_ref[0, sl] = r0
        rnd_ref[1, sl] = r1
        rnd_ref[2, sl] = r2
        m2_ref[0, sl] = -2.0 * r0
        m2_ref[1, sl] = -2.0 * r1
        m2_ref[2, sl] = -2.0 * r2
        return 0
    lax.fori_loop(c_lo, c_hi, body, 0)


def _pair_sweep(own_b_ref, base_a, opp_m2_ref, opp_sq_ref, c_lo, c_hi):
    base_b = base_a + L
    a0 = own_b_ref[0, pl.ds(base_a, L)]
    a1 = own_b_ref[1, pl.ds(base_a, L)]
    a2 = own_b_ref[2, pl.ds(base_a, L)]
    b0 = own_b_ref[0, pl.ds(base_b, L)]
    b1 = own_b_ref[1, pl.ds(base_b, L)]
    b2 = own_b_ref[2, pl.ds(base_b, L)]

    def opp_body(c, accs):
        acc_a, acc_b = accs
        w0 = opp_m2_ref[0, pl.ds(c * L, L)]
        w1 = opp_m2_ref[1, pl.ds(c * L, L)]
        w2 = opp_m2_ref[2, pl.ds(c * L, L)]
        ws = opp_sq_ref[pl.ds(c * L, L)]
        for t in range(L):
            c0 = w0[t]
            c1 = w1[t]
            c2 = w2[t]
            cs = ws[t]
            e_a = cs + (a0 * c0 + a1 * c1 + a2 * c2)
            e_b = cs + (b0 * c0 + b1 * c1 + b2 * c2)
            acc_a = jnp.minimum(acc_a, e_a)
            acc_b = jnp.minimum(acc_b, e_b)
        return acc_a, acc_b

    init = (jnp.full((L,), 1e30, jnp.float32),
            jnp.full((L,), 1e30, jnp.float32))
    return lax.fori_loop(c_lo, c_hi, opp_body, init)


def _sc_body(x_hbm, y_hbm, out1_hbm, out2_hbm,
             xorig, yorig, xb, yb, xm2, ym2, xsq, ysq, obuf, ybuf):
    c = lax.axis_index("c")
    s = lax.axis_index("s")
    wid = s * 2 + c
    b = wid // WPB
    chunk = wid % WPB

    pltpu.sync_copy(x_hbm.at[b], xorig)
    pltpu.sync_copy(y_hbm.at[b], yorig)

    _precompute(xorig, xsq, xb, xm2, C0, NC)
    _precompute(yorig, ysq, yb, ym2, 0, NC)

    xbase = R + chunk * SCPW
    m_a, m_b = _pair_sweep(xb, xbase, ym2, ysq, 0, NC)
    zero = jnp.zeros((L,), jnp.float32)
    sq_a = xsq[pl.ds(xbase, L)]
    sq_b = xsq[pl.ds(xbase + L, L)]
    obuf[...] = (jnp.maximum(m_a + sq_a, zero)
                 + jnp.maximum(m_b + sq_b, zero))
    pltpu.sync_copy(obuf, out1_hbm.at[wid])

    ybase = chunk * CH
    def ygroup(gg, _):
        base_a = ybase + gg * (2 * L)
        m_ya, m_yb = _pair_sweep(yb, base_a, xm2, xsq, C0, NC)
        ybuf[pl.ds(gg * 2 * L, L)] = m_ya + ysq[pl.ds(base_a, L)]
        ybuf[pl.ds(gg * 2 * L + L, L)] = m_yb + ysq[pl.ds(base_a + L, L)]
        return 0
    lax.fori_loop(0, CH // (2 * L), ygroup, 0)
    pltpu.sync_copy(ybuf, out2_hbm.at[wid])


def _sc_call(xT, yT):
    mesh = plsc.VectorSubcoreMesh(core_axis_name="c", subcore_axis_name="s")
    return pl.kernel(
        _sc_body,
        out_type=(jax.ShapeDtypeStruct((NW, L), jnp.float32),
                  jax.ShapeDtypeStruct((NW, CH), jnp.float32)),
        mesh=mesh,
        scratch_types=[
            pltpu.VMEM((3, N), jnp.float32),
            pltpu.VMEM((3, M), jnp.float32),
            pltpu.VMEM((3, N), jnp.float32),
            pltpu.VMEM((3, M), jnp.float32),
            pltpu.VMEM((3, N), jnp.float32),
            pltpu.VMEM((3, M), jnp.float32),
            pltpu.VMEM((N,), jnp.float32),
            pltpu.VMEM((M,), jnp.float32),
            pltpu.VMEM((L,), jnp.float32),
            pltpu.VMEM((CH,), jnp.float32),
        ],
    )(xT, yT)


def kernel(xyz1, xyz2):
    xT = jnp.transpose(xyz1, (0, 2, 1))
    yT = jnp.transpose(xyz2, (0, 2, 1))

    sc_rowsum, sc_ymin = _sc_call(xT, yT)
    tc_xsum, tc_ymin = _tc_call(xyz1, yT)

    ymin = jnp.minimum(tc_ymin.reshape(B, M), sc_ymin.reshape(B, M))
    total = (tc_xsum[0, 0] + jnp.sum(sc_rowsum)
             + jnp.sum(jnp.maximum(ymin, 0.0)))
    return total * (1.0 / (B * N))


# speedup vs baseline: 1.0025x; 1.0025x over previous
import jax
import jax.numpy as jnp
from jax import lax
from jax.experimental import pallas as pl
from jax.experimental.pallas import tpu as pltpu
from jax.experimental.pallas import tpu_sc as plsc

B, N, M = 4, 4096, 4096
SCX = 256
R = N - SCX
NW = 32
WPB = NW // B
CH = M // WPB
SCPW = SCX // WPB
L = 16
C0 = R // L
NC = M // L


def _limbs(v):
    h = v.astype(jnp.bfloat16)
    r1 = v - h.astype(jnp.float32)
    m = r1.astype(jnp.bfloat16)
    l = (r1 - m.astype(jnp.float32)).astype(jnp.bfloat16)
    return h, m, l


def _tc_body(x_ref, y_ref, xsum_ref, ymin_ref):
    b = pl.program_id(0)

    x = x_ref[0]
    y = y_ref[0]
    xb = x.astype(jnp.bfloat16)
    w = (y.astype(jnp.bfloat16)) * jnp.bfloat16(-2.0)

    xsq = jnp.sum(x * x, axis=1, keepdims=True)
    ysq = jnp.sum(y * y, axis=0, keepdims=True)
    xh, xm, xl = _limbs(xsq)
    yh, ym, yl = _limbs(ysq)

    ones_x = jnp.ones((R, 3), jnp.bfloat16)
    ones_y = jnp.ones((3, M), jnp.bfloat16)
    xmat = jnp.concatenate([xb, ones_x, xh, xm, xl], axis=1)
    wmat = jnp.concatenate([w, yh, ym, yl, ones_y], axis=0)

    g = jax.lax.dot_general(xmat, wmat, (((1,), (0,)), ((), ())),
                            preferred_element_type=jnp.float32)

    row_min = jnp.maximum(jnp.min(g, axis=1), 0.0)
    ymin_ref[0] = jnp.min(g, axis=0, keepdims=True)

    @pl.when(b == 0)
    def _zero_out():
        xsum_ref[...] = jnp.zeros((1, 1), jnp.float32)

    xsum_ref[...] = xsum_ref[...] + jnp.sum(row_min)


def _tc_call(xyz1, yT):
    return pl.pallas_call(
        _tc_body,
        grid=(B,),
        in_specs=[
            pl.BlockSpec((1, R, 3), lambda b: (b, 0, 0)),
            pl.BlockSpec((1, 3, M), lambda b: (b, 0, 0)),
        ],
        out_specs=[
            pl.BlockSpec((1, 1), lambda b: (0, 0)),
            pl.BlockSpec((1, 1, M), lambda b: (b, 0, 0)),
        ],
        out_shape=[
            jax.ShapeDtypeStruct((1, 1), jnp.float32),
            jax.ShapeDtypeStruct((B, 1, M), jnp.float32),
        ],
    )(xyz1, yT)


def _round_bf16(v):
    u = lax.bitcast_convert_type(v, jnp.int32)
    lsb = lax.shift_right_logical(u, 16) & 1
    u2 = (u + 0x7FFF + lsb) & jnp.int32(-65536)
    return lax.bitcast_convert_type(u2, jnp.float32)


def _precompute(orig_ref, sq_ref, rnd_ref, m2_ref, c_lo, c_hi):
    def body(c, _):
        sl = pl.ds(c * L, L)
        v0 = orig_ref[0, sl]
        v1 = orig_ref[1, sl]
        v2 = orig_ref[2, sl]
        sq_ref[sl] = v0 * v0 + v1 * v1 + v2 * v2
        r0 = _round_bf16(v0)
        r1 = _round_bf16(v1)
        r2 = _round_bf16(v2)
        rnd_ref[0, sl] = r0
        rnd_ref[1, sl] = r1
        rnd_ref[2, sl] = r2
        m2_ref[0, sl] = -2.0 * r0
        m2_ref[1, sl] = -2.0 * r1
        m2_ref[2, sl] = -2.0 * r2
        return 0
    lax.fori_loop(c_lo, c_hi, body, 0)


def _pair_sweep(own_b_ref, base_a, opp_m2_ref, opp_sq_ref, c_lo, c_hi):
    base_b = base_a + L
    a0 = own_b_ref[0, pl.ds(base_a, L)]
    a1 = own_b_ref[1, pl.ds(base_a, L)]
    a2 = own_b_ref[2, pl.ds(base_a, L)]
    b0 = own_b_ref[0, pl.ds(base_b, L)]
    b1 = own_b_ref[1, pl.ds(base_b, L)]
    b2 = own_b_ref[2, pl.ds(base_b, L)]

    def opp_body(c, accs):
        acc_a, acc_b = accs
        w0 = opp_m2_ref[0, pl.ds(c * L, L)]
        w1 = opp_m2_ref[1, pl.ds(c * L, L)]
        w2 = opp_m2_ref[2, pl.ds(c * L, L)]
        ws = opp_sq_ref[pl.ds(c * L, L)]
        for t in range(L):
            c0 = w0[t]
            c1 = w1[t]
            c2 = w2[t]
            cs = ws[t]
            e_a = cs + (a0 * c0 + a1 * c1 + a2 * c2)
            e_b = cs + (b0 * c0 + b1 * c1 + b2 * c2)
            acc_a = jnp.minimum(acc_a, e_a)
            acc_b = jnp.minimum(acc_b, e_b)
        return acc_a, acc_b

    init = (jnp.full((L,), 1e30, jnp.float32),
            jnp.full((L,), 1e30, jnp.float32))
    return lax.fori_loop(c_lo, c_hi, opp_body, init)


def _sc_body(x_hbm, y_hbm, out1_hbm, out2_hbm,
             xorig, yorig, xb, yb, xm2, ym2, xsq, ysq, obuf, ybuf):
    c = lax.axis_index("c")
    s = lax.axis_index("s")
    wid = s * 2 + c
    b = wid // WPB
    chunk = wid % WPB

    pltpu.sync_copy(x_hbm.at[b], xorig)
    pltpu.sync_copy(y_hbm.at[b], yorig)

    _precompute(xorig, xsq, xb, xm2, C0, NC)
    _precompute(yorig, ysq, yb, ym2, 0, NC)

    xbase = R + chunk * SCPW
    m_a, m_b = _pair_sweep(xb, xbase, ym2, ysq, 0, NC)
    zero = jnp.zeros((L,), jnp.float32)
    sq_a = xsq[pl.ds(xbase, L)]
    sq_b = xsq[pl.ds(xbase + L, L)]
    obuf[...] = (jnp.maximum(m_a + sq_a, zero)
                 + jnp.maximum(m_b + sq_b, zero))
    pltpu.sync_copy(obuf, out1_hbm.at[wid])

    ybase = chunk * CH
    def ygroup(gg, _):
        base_a = ybase + gg * (2 * L)
        m_ya, m_yb = _pair_sweep(yb, base_a, xm2, xsq, C0, NC)
        ybuf[pl.ds(gg * 2 * L, L)] = m_ya + ysq[pl.ds(base_a, L)]
        ybuf[pl.ds(gg * 2 * L + L, L)] = m_yb + ysq[pl.ds(base_a + L, L)]
        return 0
    lax.fori_loop(0, CH // (2 * L), ygroup, 0)
    pltpu.sync_copy(ybuf, out2_hbm.at[wid])


def _sc_call(xT, yT):
    mesh = plsc.VectorSubcoreMesh(core_axis_name="c", subcore_axis_name="s")
    return pl.kernel(
        _sc_body,
        out_type=(jax.ShapeDtypeStruct((NW, L), jnp.float32),
                  jax.ShapeDtypeStruct((NW, CH), jnp.float32)),
        mesh=mesh,
        scratch_types=[
            pltpu.VMEM((3, N), jnp.float32),
            pltpu.VMEM((3, M), jnp.float32),
            pltpu.VMEM((3, N), jnp.float32),
            pltpu.VMEM((3, M), jnp.float32),
            pltpu.VMEM((3, N), jnp.float32),
            pltpu.VMEM((3, M), jnp.float32),
            pltpu.VMEM((N,), jnp.float32),
            pltpu.VMEM((M,), jnp.float32),
            pltpu.VMEM((L,), jnp.float32),
            pltpu.VMEM((CH,), jnp.float32),
        ],
    )(xT, yT)


def kernel(xyz1, xyz2):
    xT = jnp.transpose(xyz1, (0, 2, 1))
    yT = jnp.transpose(xyz2, (0, 2, 1))

    tc_xsum, tc_ymin = _tc_call(xyz1, yT)
    sc_rowsum, sc_ymin = _sc_call(xT, yT)

    ymin = jnp.minimum(tc_ymin.reshape(B, M), sc_ymin.reshape(B, M))
    total = (tc_xsum[0, 0] + jnp.sum(sc_rowsum)
             + jnp.sum(jnp.maximum(ymin, 0.0)))
    return total * (1.0 / (B * N))


# speedup vs baseline: 1.0343x; 1.0317x over previous
import jax
import jax.numpy as jnp
from jax import lax
from jax.experimental import pallas as pl
from jax.experimental.pallas import tpu as pltpu
from jax.experimental.pallas import tpu_sc as plsc

B, N, M = 4, 4096, 4096
SCX = 128
R = N - SCX
NW = 32
WPB = NW // B
CH = M // WPB
SCPW = SCX // WPB
L = 16
C0 = R // L
NC = M // L


def _limbs(v):
    h = v.astype(jnp.bfloat16)
    r1 = v - h.astype(jnp.float32)
    m = r1.astype(jnp.bfloat16)
    l = (r1 - m.astype(jnp.float32)).astype(jnp.bfloat16)
    return h, m, l


def _tc_body(x_ref, y_ref, xsum_ref, ymin_ref):
    b = pl.program_id(0)

    x = x_ref[0]
    y = y_ref[0]
    xb = x.astype(jnp.bfloat16)
    w = (y.astype(jnp.bfloat16)) * jnp.bfloat16(-2.0)

    xsq = jnp.sum(x * x, axis=1, keepdims=True)
    ysq = jnp.sum(y * y, axis=0, keepdims=True)
    xh, xm, xl = _limbs(xsq)
    yh, ym, yl = _limbs(ysq)

    ones_x = jnp.ones((R, 3), jnp.bfloat16)
    ones_y = jnp.ones((3, M), jnp.bfloat16)
    xmat = jnp.concatenate([xb, ones_x, xh, xm, xl], axis=1)
    wmat = jnp.concatenate([w, yh, ym, yl, ones_y], axis=0)

    g = jax.lax.dot_general(xmat, wmat, (((1,), (0,)), ((), ())),
                            preferred_element_type=jnp.float32)

    row_min = jnp.maximum(jnp.min(g, axis=1), 0.0)
    ymin_ref[0] = jnp.min(g, axis=0, keepdims=True)

    @pl.when(b == 0)
    def _zero_out():
        xsum_ref[...] = jnp.zeros((1, 1), jnp.float32)

    xsum_ref[...] = xsum_ref[...] + jnp.sum(row_min)


def _tc_call(xyz1, yT):
    return pl.pallas_call(
        _tc_body,
        grid=(B,),
        in_specs=[
            pl.BlockSpec((1, R, 3), lambda b: (b, 0, 0)),
            pl.BlockSpec((1, 3, M), lambda b: (b, 0, 0)),
        ],
        out_specs=[
            pl.BlockSpec((1, 1), lambda b: (0, 0)),
            pl.BlockSpec((1, 1, M), lambda b: (b, 0, 0)),
        ],
        out_shape=[
            jax.ShapeDtypeStruct((1, 1), jnp.float32),
            jax.ShapeDtypeStruct((B, 1, M), jnp.float32),
        ],
    )(xyz1, yT)


def _round_bf16(v):
    u = lax.bitcast_convert_type(v, jnp.int32)
    lsb = lax.shift_right_logical(u, 16) & 1
    u2 = (u + 0x7FFF + lsb) & jnp.int32(-65536)
    return lax.bitcast_convert_type(u2, jnp.float32)


def _precompute(orig_ref, sq_ref, rnd_ref, m2_ref, c_lo, c_hi):
    def body(c, _):
        sl = pl.ds(c * L, L)
        v0 = orig_ref[0, sl]
        v1 = orig_ref[1, sl]
        v2 = orig_ref[2, sl]
        sq_ref[sl] = v0 * v0 + v1 * v1 + v2 * v2
        r0 = _round_bf16(v0)
        r1 = _round_bf16(v1)
        r2 = _round_bf16(v2)
        rnd_ref[0, sl] = r0
        rnd_ref[1, sl] = r1
        rnd_ref[2, sl] = r2
        m2_ref[0, sl] = -2.0 * r0
        m2_ref[1, sl] = -2.0 * r1
        m2_ref[2, sl] = -2.0 * r2
        return 0
    lax.fori_loop(c_lo, c_hi, body, 0)


def _pair_sweep(own_b_ref, base_a, opp_m2_ref, opp_sq_ref, c_lo, c_hi):
    base_b = base_a + L
    a0 = own_b_ref[0, pl.ds(base_a, L)]
    a1 = own_b_ref[1, pl.ds(base_a, L)]
    a2 = own_b_ref[2, pl.ds(base_a, L)]
    b0 = own_b_ref[0, pl.ds(base_b, L)]
    b1 = own_b_ref[1, pl.ds(base_b, L)]
    b2 = own_b_ref[2, pl.ds(base_b, L)]

    def opp_body(c, accs):
        acc_a, acc_b = accs
        w0 = opp_m2_ref[0, pl.ds(c * L, L)]
        w1 = opp_m2_ref[1, pl.ds(c * L, L)]
        w2 = opp_m2_ref[2, pl.ds(c * L, L)]
        ws = opp_sq_ref[pl.ds(c * L, L)]
        for t in range(L):
            c0 = w0[t]
            c1 = w1[t]
            c2 = w2[t]
            cs = ws[t]
            e_a = cs + (a0 * c0 + a1 * c1 + a2 * c2)
            e_b = cs + (b0 * c0 + b1 * c1 + b2 * c2)
            acc_a = jnp.minimum(acc_a, e_a)
            acc_b = jnp.minimum(acc_b, e_b)
        return acc_a, acc_b

    init = (jnp.full((L,), 1e30, jnp.float32),
            jnp.full((L,), 1e30, jnp.float32))
    return lax.fori_loop(c_lo, c_hi, opp_body, init)


def _pair_sweep1(own_b_ref, base_a, opp_m2_ref, opp_sq_ref, c_lo, c_hi):
    a0 = own_b_ref[0, pl.ds(base_a, L)]
    a1 = own_b_ref[1, pl.ds(base_a, L)]
    a2 = own_b_ref[2, pl.ds(base_a, L)]

    def opp_body(c, acc):
        w0 = opp_m2_ref[0, pl.ds(c * L, L)]
        w1 = opp_m2_ref[1, pl.ds(c * L, L)]
        w2 = opp_m2_ref[2, pl.ds(c * L, L)]
        ws = opp_sq_ref[pl.ds(c * L, L)]
        for t in range(L):
            e_a = ws[t] + (a0 * w0[t] + a1 * w1[t] + a2 * w2[t])
            acc = jnp.minimum(acc, e_a)
        return acc

    return lax.fori_loop(c_lo, c_hi, opp_body,
                         jnp.full((L,), 1e30, jnp.float32))


def _sc_body(x_hbm, y_hbm, out1_hbm, out2_hbm,
             xorig, yorig, xb, yb, xm2, ym2, xsq, ysq, obuf, ybuf):
    c = lax.axis_index("c")
    s = lax.axis_index("s")
    wid = s * 2 + c
    b = wid // WPB
    chunk = wid % WPB

    pltpu.sync_copy(x_hbm.at[b], xorig)
    pltpu.sync_copy(y_hbm.at[b], yorig)

    _precompute(xorig, xsq, xb, xm2, C0, NC)
    _precompute(yorig, ysq, yb, ym2, 0, NC)

    xbase = R + chunk * SCPW
    zero = jnp.zeros((L,), jnp.float32)
    if SCPW == 32:
        m_a, m_b = _pair_sweep(xb, xbase, ym2, ysq, 0, NC)
        sq_a = xsq[pl.ds(xbase, L)]
        sq_b = xsq[pl.ds(xbase + L, L)]
        obuf[...] = (jnp.maximum(m_a + sq_a, zero)
                     + jnp.maximum(m_b + sq_b, zero))
    else:
        m_a = _pair_sweep1(xb, xbase, ym2, ysq, 0, NC)
        obuf[...] = jnp.maximum(m_a + xsq[pl.ds(xbase, L)], zero)
    pltpu.sync_copy(obuf, out1_hbm.at[wid])

    ybase = chunk * CH
    def ygroup(gg, _):
        base_a = ybase + gg * (2 * L)
        m_ya, m_yb = _pair_sweep(yb, base_a, xm2, xsq, C0, NC)
        ybuf[pl.ds(gg * 2 * L, L)] = m_ya + ysq[pl.ds(base_a, L)]
        ybuf[pl.ds(gg * 2 * L + L, L)] = m_yb + ysq[pl.ds(base_a + L, L)]
        return 0
    lax.fori_loop(0, CH // (2 * L), ygroup, 0)
    pltpu.sync_copy(ybuf, out2_hbm.at[wid])


def _sc_call(xT, yT):
    mesh = plsc.VectorSubcoreMesh(core_axis_name="c", subcore_axis_name="s")
    return pl.kernel(
        _sc_body,
        out_type=(jax.ShapeDtypeStruct((NW, L), jnp.float32),
                  jax.ShapeDtypeStruct((NW, CH), jnp.float32)),
        mesh=mesh,
        scratch_types=[
            pltpu.VMEM((3, N), jnp.float32),
            pltpu.VMEM((3, M), jnp.float32),
            pltpu.VMEM((3, N), jnp.float32),
            pltpu.VMEM((3, M), jnp.float32),
            pltpu.VMEM((3, N), jnp.float32),
            pltpu.VMEM((3, M), jnp.float32),
            pltpu.VMEM((N,), jnp.float32),
            pltpu.VMEM((M,), jnp.float32),
            pltpu.VMEM((L,), jnp.float32),
            pltpu.VMEM((CH,), jnp.float32),
        ],
    )(xT, yT)


def kernel(xyz1, xyz2):
    xT = jnp.transpose(xyz1, (0, 2, 1))
    yT = jnp.transpose(xyz2, (0, 2, 1))

    tc_xsum, tc_ymin = _tc_call(xyz1, yT)
    sc_rowsum, sc_ymin = _sc_call(xT, yT)

    ymin = jnp.minimum(tc_ymin.reshape(B, M), sc_ymin.reshape(B, M))
    total = (tc_xsum[0, 0] + jnp.sum(sc_rowsum)
             + jnp.sum(jnp.maximum(ymin, 0.0)))
    return total * (1.0 / (B * N))


# speedup vs baseline: 1.0371x; 1.0027x over previous
import jax
import jax.numpy as jnp
from jax import lax
from jax.experimental import pallas as pl
from jax.experimental.pallas import tpu as pltpu
from jax.experimental.pallas import tpu_sc as plsc

B, N, M = 4, 4096, 4096
SCX = 128
R = N - SCX
NW = 32
WPB = NW // B
CH = M // WPB
SCPW = SCX // WPB
L = 16
C0 = R // L
NC = M // L


def _limbs(v):
    h = v.astype(jnp.bfloat16)
    r1 = v - h.astype(jnp.float32)
    m = r1.astype(jnp.bfloat16)
    l = (r1 - m.astype(jnp.float32)).astype(jnp.bfloat16)
    return h, m, l


def _tc_body(x_ref, y_ref, xsum_ref, ymin_ref):
    b = pl.program_id(0)

    x = x_ref[0]
    y = y_ref[0]
    xb = x.astype(jnp.bfloat16)
    w = (y.astype(jnp.bfloat16)) * jnp.bfloat16(-2.0)

    xsq = jnp.sum(x * x, axis=1, keepdims=True)
    ysq = jnp.sum(y * y, axis=0, keepdims=True)
    xh, xm, xl = _limbs(xsq)
    yh, ym, yl = _limbs(ysq)

    ones_x = jnp.ones((R, 3), jnp.bfloat16)
    ones_y = jnp.ones((3, M), jnp.bfloat16)
    xmat = jnp.concatenate([xb, ones_x, xh, xm, xl], axis=1)
    wmat = jnp.concatenate([w, yh, ym, yl, ones_y], axis=0)

    g = jax.lax.dot_general(xmat, wmat, (((1,), (0,)), ((), ())),
                            preferred_element_type=jnp.float32)

    row_min = jnp.maximum(jnp.min(g, axis=1), 0.0)
    ymin_ref[0] = jnp.min(g, axis=0, keepdims=True)

    @pl.when(b == 0)
    def _zero_out():
        xsum_ref[...] = jnp.zeros((1, 1), jnp.float32)

    xsum_ref[...] = xsum_ref[...] + jnp.sum(row_min)


def _tc_call(xyz1, yT):
    return pl.pallas_call(
        _tc_body,
        grid=(B,),
        in_specs=[
            pl.BlockSpec((1, R, 3), lambda b: (b, 0, 0)),
            pl.BlockSpec((1, 3, M), lambda b: (b, 0, 0)),
        ],
        out_specs=[
            pl.BlockSpec((1, 1), lambda b: (0, 0)),
            pl.BlockSpec((1, 1, M), lambda b: (b, 0, 0)),
        ],
        out_shape=[
            jax.ShapeDtypeStruct((1, 1), jnp.float32),
            jax.ShapeDtypeStruct((B, 1, M), jnp.float32),
        ],
    )(xyz1, yT)


def _round_bf16(v):
    u = lax.bitcast_convert_type(v, jnp.int32)
    lsb = lax.shift_right_logical(u, 16) & 1
    u2 = (u + 0x7FFF + lsb) & jnp.int32(-65536)
    return lax.bitcast_convert_type(u2, jnp.float32)


def _precompute(orig_ref, sq_ref, rnd_ref, m2_ref, c_lo, c_hi):
    def body(c, _):
        sl = pl.ds(c * L, L)
        v0 = orig_ref[0, sl]
        v1 = orig_ref[1, sl]
        v2 = orig_ref[2, sl]
        sq_ref[sl] = v0 * v0 + v1 * v1 + v2 * v2
        r0 = _round_bf16(v0)
        r1 = _round_bf16(v1)
        r2 = _round_bf16(v2)
        rnd_ref[0, sl] = r0
        rnd_ref[1, sl] = r1
        rnd_ref[2, sl] = r2
        m2_ref[0, sl] = -2.0 * r0
        m2_ref[1, sl] = -2.0 * r1
        m2_ref[2, sl] = -2.0 * r2
        return 0
    lax.fori_loop(c_lo, c_hi, body, 0)


def _pair_sweep(own_b_ref, base_a, opp_m2_ref, opp_sq_ref, c_lo, c_hi):
    base_b = base_a + L
    a0 = own_b_ref[0, pl.ds(base_a, L)]
    a1 = own_b_ref[1, pl.ds(base_a, L)]
    a2 = own_b_ref[2, pl.ds(base_a, L)]
    b0 = own_b_ref[0, pl.ds(base_b, L)]
    b1 = own_b_ref[1, pl.ds(base_b, L)]
    b2 = own_b_ref[2, pl.ds(base_b, L)]

    def opp_body(c, accs):
        acc_a, acc_b = accs
        w0 = opp_m2_ref[0, pl.ds(c * L, L)]
        w1 = opp_m2_ref[1, pl.ds(c * L, L)]
        w2 = opp_m2_ref[2, pl.ds(c * L, L)]
        ws = opp_sq_ref[pl.ds(c * L, L)]
        for t in range(L):
            c0 = w0[t]
            c1 = w1[t]
            c2 = w2[t]
            cs = ws[t]
            e_a = cs + (a0 * c0 + a1 * c1 + a2 * c2)
            e_b = cs + (b0 * c0 + b1 * c1 + b2 * c2)
            acc_a = jnp.minimum(acc_a, e_a)
            acc_b = jnp.minimum(acc_b, e_b)
        return acc_a, acc_b

    init = (jnp.full((L,), 1e30, jnp.float32),
            jnp.full((L,), 1e30, jnp.float32))
    return lax.fori_loop(c_lo, c_hi, opp_body, init)


def _pair_sweep1(own_b_ref, base_a, opp_m2_ref, opp_sq_ref, c_lo, c_hi):
    a0 = own_b_ref[0, pl.ds(base_a, L)]
    a1 = own_b_ref[1, pl.ds(base_a, L)]
    a2 = own_b_ref[2, pl.ds(base_a, L)]

    def opp_body(c, acc):
        w0 = opp_m2_ref[0, pl.ds(c * L, L)]
        w1 = opp_m2_ref[1, pl.ds(c * L, L)]
        w2 = opp_m2_ref[2, pl.ds(c * L, L)]
        ws = opp_sq_ref[pl.ds(c * L, L)]
        for t in range(L):
            e_a = ws[t] + (a0 * w0[t] + a1 * w1[t] + a2 * w2[t])
            acc = jnp.minimum(acc, e_a)
        return acc

    return lax.fori_loop(c_lo, c_hi, opp_body,
                         jnp.full((L,), 1e30, jnp.float32))


def _sc_body(x_hbm, y_hbm, out1_hbm, out2_hbm,
             xorig, yorig, xb, yb, xm2, ym2, xsq, ysq, obuf, ybuf):
    c = lax.axis_index("c")
    s = lax.axis_index("s")
    wid = s * 2 + c
    b = wid // WPB
    chunk = wid % WPB

    pltpu.sync_copy(x_hbm.at[b, :, pl.ds(R, SCX)], xorig)
    pltpu.sync_copy(y_hbm.at[b], yorig)

    _precompute(xorig, xsq, xb, xm2, 0, SCX // L)
    _precompute(yorig, ysq, yb, ym2, 0, NC)

    xbase = chunk * SCPW
    zero = jnp.zeros((L,), jnp.float32)
    if SCPW == 32:
        m_a, m_b = _pair_sweep(xb, xbase, ym2, ysq, 0, NC)
        sq_a = xsq[pl.ds(xbase, L)]
        sq_b = xsq[pl.ds(xbase + L, L)]
        obuf[...] = (jnp.maximum(m_a + sq_a, zero)
                     + jnp.maximum(m_b + sq_b, zero))
    else:
        m_a = _pair_sweep1(xb, xbase, ym2, ysq, 0, NC)
        obuf[...] = jnp.maximum(m_a + xsq[pl.ds(xbase, L)], zero)
    pltpu.sync_copy(obuf, out1_hbm.at[wid])

    ybase = chunk * CH
    def ygroup(gg, _):
        base_a = ybase + gg * (2 * L)
        m_ya, m_yb = _pair_sweep(yb, base_a, xm2, xsq, 0, SCX // L)
        ybuf[pl.ds(gg * 2 * L, L)] = m_ya + ysq[pl.ds(base_a, L)]
        ybuf[pl.ds(gg * 2 * L + L, L)] = m_yb + ysq[pl.ds(base_a + L, L)]
        return 0
    lax.fori_loop(0, CH // (2 * L), ygroup, 0)
    pltpu.sync_copy(ybuf, out2_hbm.at[wid])


def _sc_call(xT, yT):
    mesh = plsc.VectorSubcoreMesh(core_axis_name="c", subcore_axis_name="s")
    return pl.kernel(
        _sc_body,
        out_type=(jax.ShapeDtypeStruct((NW, L), jnp.float32),
                  jax.ShapeDtypeStruct((NW, CH), jnp.float32)),
        mesh=mesh,
        scratch_types=[
            pltpu.VMEM((3, SCX), jnp.float32),
            pltpu.VMEM((3, M), jnp.float32),
            pltpu.VMEM((3, SCX), jnp.float32),
            pltpu.VMEM((3, M), jnp.float32),
            pltpu.VMEM((3, SCX), jnp.float32),
            pltpu.VMEM((3, M), jnp.float32),
            pltpu.VMEM((SCX,), jnp.float32),
            pltpu.VMEM((M,), jnp.float32),
            pltpu.VMEM((L,), jnp.float32),
            pltpu.VMEM((CH,), jnp.float32),
        ],
    )(xT, yT)


def kernel(xyz1, xyz2):
    xT = jnp.transpose(xyz1, (0, 2, 1))
    yT = jnp.transpose(xyz2, (0, 2, 1))

    tc_xsum, tc_ymin = _tc_call(xyz1, yT)
    sc_rowsum, sc_ymin = _sc_call(xT, yT)

    ymin = jnp.minimum(tc_ymin.reshape(B, M), sc_ymin.reshape(B, M))
    total = (tc_xsum[0, 0] + jnp.sum(sc_rowsum)
             + jnp.sum(jnp.maximum(ymin, 0.0)))
    return total * (1.0 / (B * N))
